# Initial kernel scaffold; baseline (speedup 1.0000x reference)
import jax
import jax.numpy as jnp
from jax import lax
from jax.experimental import pallas as pl
from jax.experimental.pallas import tpu as pltpu
from jax.experimental.pallas import tpu_sc as plsc

B = 4096
L = 200
T = B * L
D = 32
NC = 2
NS = 16
NW = NC * NS
TOK_PER_W = T // NW
BLK = 128
K = 10
C = K * BLK
NCHUNK = TOK_PER_W // C
NBLK_PER_W = TOK_PER_W // BLK


def _emb_body(wi_hbm, ti_hbm, wt_hbm, tt_hbm, out_hbm,
              idx_w, idx_t, w_v, t_v, sem):
    wid = lax.axis_index("s") * NC + lax.axis_index("c")
    blk0 = wid * NBLK_PER_W
    tok0 = wid * TOK_PER_W

    def chunk_body(i, carry):
        b = blk0 + i * K
        tok = tok0 + i * C
        pltpu.sync_copy(wi_hbm.at[pl.ds(b, K), :], idx_w)
        pltpu.sync_copy(ti_hbm.at[pl.ds(b, K), :], idx_t)
        handles = []
        for j in range(K):
            handles.append(pltpu.async_copy(
                wt_hbm.at[idx_w.at[j]], w_v.at[pl.ds(j * BLK, BLK), :], sem))
            handles.append(pltpu.async_copy(
                tt_hbm.at[idx_t.at[j]], t_v.at[pl.ds(j * BLK, BLK), :], sem))
        for h in handles:
            h.wait()
        pltpu.sync_copy(w_v, out_hbm.at[pl.ds(tok, C), pl.ds(0, D)])
        pltpu.sync_copy(t_v, out_hbm.at[pl.ds(tok, C), pl.ds(D, D)])
        return carry

    lax.fori_loop(0, NCHUNK, chunk_body, 0)


def kernel(words_token_ids, tags_token_ids, words_table, tags_table):
    wi = words_token_ids.reshape(T // BLK, BLK)
    ti = tags_token_ids.reshape(T // BLK, BLK)
    mesh = plsc.VectorSubcoreMesh(core_axis_name="c", subcore_axis_name="s")
    out = pl.kernel(
        _emb_body,
        mesh=mesh,
        out_type=jax.ShapeDtypeStruct((T, 2 * D), jnp.float32),
        scratch_types=[
            pltpu.VMEM((K, BLK), jnp.int32),
            pltpu.VMEM((K, BLK), jnp.int32),
            pltpu.VMEM((C, D), jnp.float32),
            pltpu.VMEM((C, D), jnp.float32),
            pltpu.SemaphoreType.DMA,
        ],
    )(wi, ti, words_table, tags_table)
    return out.reshape(B, L, 2 * D)


# speedup vs baseline: 3.0080x; 3.0080x over previous
import jax
import jax.numpy as jnp
from jax import lax
from jax.experimental import pallas as pl
from jax.experimental.pallas import tpu as pltpu
from jax.experimental.pallas import tpu_sc as plsc

B = 4096
L = 200
T = B * L
D = 32
NC = 2
NS = 16
NW = NC * NS
TOK_PER_W = T // NW
BLK = 128
K = 8
C = K * BLK
NCHUNK = TOK_PER_W // C
NBLK_PER_W = TOK_PER_W // BLK


def _emb_body(wi_hbm, ti_hbm, wt_hbm, tt_hbm, out_hbm,
              idx_w, idx_t, w_v, t_v, sem):
    wid = lax.axis_index("s") * NC + lax.axis_index("c")
    blk0 = wid * NBLK_PER_W

    def chunk_body(i, carry):
        b = blk0 + i * K
        tok = b * BLK
        pltpu.sync_copy(wi_hbm.at[pl.ds(b, K), :], idx_w)
        pltpu.sync_copy(ti_hbm.at[pl.ds(b, K), :], idx_t)
        gathers = []
        for j in range(K):
            gathers.append(pltpu.async_copy(
                wt_hbm.at[idx_w.at[j]], w_v.at[pl.ds(j * BLK, BLK), :], sem))
            gathers.append(pltpu.async_copy(
                tt_hbm.at[idx_t.at[j]], t_v.at[pl.ds(j * BLK, BLK), :], sem))
        for h in gathers:
            h.wait()
        pltpu.sync_copy(w_v, out_hbm.at[pl.ds(tok, C), pl.ds(0, D)])
        pltpu.sync_copy(t_v, out_hbm.at[pl.ds(tok, C), pl.ds(D, D)])
        return carry

    lax.fori_loop(0, NCHUNK, chunk_body, 0)


def kernel(words_token_ids, tags_token_ids, words_table, tags_table):
    wi = words_token_ids.reshape(T // BLK, BLK)
    ti = tags_token_ids.reshape(T // BLK, BLK)
    mesh = plsc.VectorSubcoreMesh(core_axis_name="c", subcore_axis_name="s")
    out = pl.kernel(
        _emb_body,
        mesh=mesh,
        compiler_params=pltpu.CompilerParams(use_tc_tiling_on_sc=False),
        out_type=jax.ShapeDtypeStruct((T, 2 * D), jnp.float32),
        scratch_types=[
            pltpu.VMEM((K, BLK), jnp.int32),
            pltpu.VMEM((K, BLK), jnp.int32),
            pltpu.VMEM((C, D), jnp.float32),
            pltpu.VMEM((C, D), jnp.float32),
            pltpu.SemaphoreType.DMA,
        ],
    )(wi, ti, words_table, tags_table)
    return out.reshape(B, L, 2 * D)
